# Initial kernel scaffold; baseline (speedup 1.0000x reference)
import functools

import jax
import jax.numpy as jnp
from jax import lax
from jax.experimental import pallas as pl
from jax.experimental.pallas import tpu as pltpu
from jax.experimental.pallas import tpu_sc as plsc

NUM_ROWS = 1000000
DIM = 32

B_TOTAL = 16384 * 200
IDX_MINOR = 128
K = 8
CHUNK = K * IDX_MINOR
NW = 32
B_PER_W = B_TOTAL // NW
ROWS_PER_W = B_PER_W // IDX_MINOR
CHUNKS_PER_W = B_PER_W // CHUNK


def _body(idx_hbm, w_hbm, out_hbm, idx_v, rows_v, sem):
    wid = lax.axis_index("s") * 2 + lax.axis_index("c")
    row0 = wid * ROWS_PER_W

    def chunk(g, carry):
        r = row0 + g * K
        pltpu.sync_copy(idx_hbm.at[pl.ds(r, K), :], idx_v)
        copies = [
            pltpu.async_copy(
                w_hbm.at[idx_v.at[j]],
                rows_v.at[pl.ds(j * IDX_MINOR, IDX_MINOR), :],
                sem,
            )
            for j in range(K)
        ]
        for c in copies:
            c.wait()
        pltpu.sync_copy(rows_v, out_hbm.at[pl.ds(r * IDX_MINOR, CHUNK), :])
        return carry

    lax.fori_loop(0, CHUNKS_PER_W, chunk, 0)


_mesh = plsc.VectorSubcoreMesh(core_axis_name="c", subcore_axis_name="s")

_gather = functools.partial(
    pl.kernel,
    out_type=jax.ShapeDtypeStruct((B_TOTAL, DIM), jnp.float32),
    mesh=_mesh,
    scratch_types=[
        pltpu.VMEM((K, IDX_MINOR), jnp.int32),
        pltpu.VMEM((CHUNK, DIM), jnp.float32),
        pltpu.SemaphoreType.DMA,
    ],
)(_body)


def kernel(x, weight):
    idx = x.reshape(B_TOTAL // IDX_MINOR, IDX_MINOR).astype(jnp.int32)
    out = _gather(idx, weight)
    return out.reshape(*x.shape, DIM)


# speedup vs baseline: 4.8074x; 4.8074x over previous
import functools

import jax
import jax.numpy as jnp
from jax import lax
from jax.experimental import pallas as pl
from jax.experimental.pallas import tpu as pltpu
from jax.experimental.pallas import tpu_sc as plsc

NUM_ROWS = 1000000
DIM = 32

B_TOTAL = 16384 * 200
IDX_MINOR = 128
K = 8
CHUNK = K * IDX_MINOR
NW = 32
B_PER_W = B_TOTAL // NW
ROWS_PER_W = B_PER_W // IDX_MINOR
CHUNKS_PER_W = B_PER_W // CHUNK


def _body(idx_hbm, w_hbm, out_hbm, idx_v, rows_v, sem):
    wid = lax.axis_index("s") * 2 + lax.axis_index("c")
    row0 = wid * ROWS_PER_W

    def chunk(g, carry):
        r = row0 + g * K
        pltpu.sync_copy(idx_hbm.at[pl.ds(r, K), :], idx_v)
        copies = [
            pltpu.async_copy(
                w_hbm.at[idx_v.at[j]],
                rows_v.at[pl.ds(j * IDX_MINOR, IDX_MINOR), :],
                sem,
            )
            for j in range(K)
        ]
        for c in copies:
            c.wait()
        pltpu.sync_copy(rows_v, out_hbm.at[pl.ds(r * IDX_MINOR, CHUNK), :])
        return carry

    lax.fori_loop(0, CHUNKS_PER_W, chunk, 0)


_mesh = plsc.VectorSubcoreMesh(core_axis_name="c", subcore_axis_name="s")

_gather = functools.partial(
    pl.kernel,
    out_type=jax.ShapeDtypeStruct((B_TOTAL, DIM), jnp.float32),
    mesh=_mesh,
    scratch_types=[
        pltpu.VMEM((K, IDX_MINOR), jnp.int32),
        pltpu.VMEM((CHUNK, DIM), jnp.float32),
        pltpu.SemaphoreType.DMA,
    ],
    compiler_params=pltpu.CompilerParams(use_tc_tiling_on_sc=False),
)(_body)


def kernel(x, weight):
    idx = x.reshape(B_TOTAL // IDX_MINOR, IDX_MINOR).astype(jnp.int32)
    out = _gather(idx, weight)
    return out.reshape(*x.shape, DIM)
